# baseline (speedup 1.0000x reference)
import jax
import jax.numpy as jnp
from jax import lax
from jax.experimental import pallas as pl
from jax.experimental.pallas import tpu as pltpu
from jax.experimental.pallas import tpu_sc as plsc

N_NODES = 10000
N_EDGES = 320000
D_FEAT = 128
D_HALF = 64

N_PAD = 10240
N_SUBCORES = 16
ROWS_PER_TILE = N_PAD // N_SUBCORES
CHUNK_E = 128
N_CHUNKS = 160
E_PAD = N_CHUNKS * CHUNK_E * N_SUBCORES
ROW_CHUNK = 16
N_ROW_CHUNKS = ROWS_PER_TILE // ROW_CHUNK


def _zero_rows(zbuf, buf, r0):
    def body(q, _):
        pltpu.sync_copy(zbuf, buf.at[pl.ds(r0 + q * ROW_CHUNK, ROW_CHUNK)])
        return _

    lax.fori_loop(0, N_ROW_CHUNKS, body, None)


def _zero_fill(ref, n_rows):
    zv = jnp.zeros((16,), jnp.float32)

    def body(j, _):
        for d in range(D_HALF // 16):
            ref[j, pl.ds(16 * d, 16)] = zv
        return _

    lax.fori_loop(0, n_rows, body, None)


def _scale_rows(wbuf, rows, scaled):
    def group_body(g, _):
        wgrp = wbuf[pl.ds(g * 16, 16)]
        for j16 in range(16):
            j = g * 16 + j16
            wvec = wgrp.at[jnp.full((16,), j16, jnp.int32)].get(
                mode="promise_in_bounds")
            for d in range(D_HALF // 16):
                sl = pl.ds(16 * d, 16)
                scaled[j, sl] = rows[j, sl] * wvec
        return _

    lax.fori_loop(0, CHUNK_E // 16, group_body, None)


def _copy_idx(ebuf, srcc, dstc):
    def body(q, _):
        sl = pl.ds(16 * q, 16)
        srcc[sl] = ebuf[0, sl]
        dstc[sl] = ebuf[1, sl]
        return _

    lax.fori_loop(0, CHUNK_E // 16, body, None)


def _hop(src_buf, acc_buf, epack, wpack, sid, bufs):
    (ebuf, wbuf, rows, scaled, srcc, dstc, se, sw, sg, ss) = bufs
    c_base = sid * N_CHUNKS

    def load_e(s, c):
        return pltpu.async_copy(epack.at[c_base + c], ebuf[s], se[s])

    def load_w(s, c):
        return pltpu.async_copy(wpack.at[c_base + c], wbuf[s], sw[s])

    def stage_a(sr, sd, c):
        pltpu.make_async_copy(epack.at[c_base], ebuf[sr], se[sr]).wait()
        _copy_idx(ebuf[sr], srcc[sr], dstc[sd])
        load_e(sr, jnp.minimum(c + 2, N_CHUNKS - 2 + sr))
        pltpu.async_copy(src_buf.at[srcc[sr]], rows[sr], sg[sr])

    def stage_b(sr, sd, c, first):
        pltpu.make_async_copy(src_buf.at[srcc[sr]], rows[sr], sg[sr]).wait()
        pltpu.make_async_copy(wpack.at[c_base], wbuf[sr], sw[sr]).wait()
        if not first:
            pltpu.make_async_copy(scaled[sr], acc_buf.at[dstc[sd]],
                                  ss[sr]).wait()
        _scale_rows(wbuf[sr], rows[sr], scaled[sr])
        load_w(sr, jnp.minimum(c + 2, N_CHUNKS - 2 + sr))
        pltpu.async_copy(scaled[sr], acc_buf.at[dstc[sd]], ss[sr], add=True)

    load_e(0, 0)
    load_w(0, 0)
    load_e(1, 1)
    load_w(1, 1)
    stage_a(0, 0, 0)
    stage_a(1, 1, 1)
    stage_b(0, 0, 0, first=True)
    stage_a(0, 2, 2)
    stage_b(1, 1, 1, first=True)
    stage_a(1, 3, 3)
    stage_b(0, 2, 2, first=False)

    def body(t, _):
        c0 = t * 4
        stage_a(0, 0, c0)
        stage_b(1, 3, c0 - 1, first=False)
        stage_a(1, 1, c0 + 1)
        stage_b(0, 0, c0, first=False)
        stage_a(0, 2, c0 + 2)
        stage_b(1, 1, c0 + 1, first=False)
        stage_a(1, 3, c0 + 3)
        stage_b(0, 2, c0 + 2, first=False)
        return _

    lax.fori_loop(1, N_CHUNKS // 4, body, None)

    stage_b(1, 3, N_CHUNKS - 1, first=False)
    for s in range(2):
        pltpu.make_async_copy(scaled[s], acc_buf.at[dstc[s]], ss[s]).wait()
        pltpu.make_async_copy(epack.at[c_base], ebuf[s], se[s]).wait()
        pltpu.make_async_copy(wpack.at[c_base], wbuf[s], sw[s]).wait()


def _combine_chunk(t0c, t1c, t2c, t3c, lowc, highc, coefs):
    c2l, c1l, c0l, c2h, c1h, c0h = coefs

    def body(j, _):
        for d in range(D_HALF // 16):
            sl = pl.ds(16 * d, 16)
            v0 = t0c[j, sl]
            v1 = t1c[j, sl]
            v2 = t2c[j, sl]
            v3 = t3c[j, sl]
            lowc[j, sl] = v3 + c2l * v2 + c1l * v1 + c0l * v0
            highc[j, sl] = v3 + c2h * v2 + c1h * v1 + c0h * v0
        return _

    lax.fori_loop(0, ROW_CHUNK, body, None)


def _filter_kernel(x_pair, epack, wpack, da_hbm,
                   low_hbm, high_hbm, tx1_hbm, tx2_hbm,
                   buf_a, buf_b,
                   e0, e1, w0, w1, r0b, r1b, p0b, p1b,
                   sc0, sc1, dc0, dc1, dc2, dc3,
                   zbuf, t0c, t1c, t2c, t3c, lowc, highc, dav,
                   se0, se1, sw0, sw1, sg0, sg1, ss0, ss1):
    cid = lax.axis_index("c")
    sid = lax.axis_index("s")
    r0 = sid * ROWS_PER_TILE
    bufs = ((e0, e1), (w0, w1), (r0b, r1b), (p0b, p1b),
            (sc0, sc1), (dc0, dc1, dc2, dc3),
            (se0, se1), (sw0, sw1), (sg0, sg1), (ss0, ss1))

    pltpu.sync_copy(x_pair.at[cid, pl.ds(r0, ROWS_PER_TILE)],
                    buf_a.at[pl.ds(r0, ROWS_PER_TILE)])
    pltpu.sync_copy(da_hbm, dav)
    _zero_fill(zbuf, ROW_CHUNK)
    _zero_rows(zbuf, buf_b, r0)
    plsc.subcore_barrier()

    _hop(buf_a, buf_b, epack, wpack, sid, bufs)
    plsc.subcore_barrier()

    pltpu.sync_copy(buf_b.at[pl.ds(r0, ROWS_PER_TILE)],
                    tx1_hbm.at[cid, pl.ds(r0, ROWS_PER_TILE)])
    _zero_rows(zbuf, buf_a, r0)
    plsc.subcore_barrier()

    _hop(buf_b, buf_a, epack, wpack, sid, bufs)
    plsc.subcore_barrier()

    pltpu.sync_copy(buf_a.at[pl.ds(r0, ROWS_PER_TILE)],
                    tx2_hbm.at[cid, pl.ds(r0, ROWS_PER_TILE)])
    _zero_rows(zbuf, buf_b, r0)
    plsc.subcore_barrier()

    _hop(buf_a, buf_b, epack, wpack, sid, bufs)
    plsc.subcore_barrier()

    davec = dav[pl.ds(0, 16)]
    d = davec[0]
    av = davec[1]
    d2 = d * d
    c2l = -3.0 * d - av
    c1l = 3.0 * d2 + 2.0 * d * av
    c0l = -(d2 * d + d2 * av)
    c2h = -3.0 * d + av
    c1h = 3.0 * d2 - 2.0 * d * av
    c0h = d2 * av - d2 * d
    coefs = (c2l, c1l, c0l, c2h, c1h, c0h)

    def final_body(q, _):
        rq = r0 + q * ROW_CHUNK
        pltpu.sync_copy(x_pair.at[cid, pl.ds(rq, ROW_CHUNK)], t0c)
        pltpu.sync_copy(tx1_hbm.at[cid, pl.ds(rq, ROW_CHUNK)], t1c)
        pltpu.sync_copy(tx2_hbm.at[cid, pl.ds(rq, ROW_CHUNK)], t2c)
        pltpu.sync_copy(buf_b.at[pl.ds(rq, ROW_CHUNK)], t3c)
        _combine_chunk(t0c, t1c, t2c, t3c, lowc, highc, coefs)
        pltpu.sync_copy(lowc, low_hbm.at[cid, pl.ds(rq, ROW_CHUNK)])
        pltpu.sync_copy(highc, high_hbm.at[cid, pl.ds(rq, ROW_CHUNK)])
        return _

    lax.fori_loop(0, N_ROW_CHUNKS, final_body, None)


@jax.jit
def _run(x_pair, epack, wpack, da):
    mesh = plsc.VectorSubcoreMesh(core_axis_name="c", subcore_axis_name="s")
    f = pl.kernel(
        _filter_kernel,
        mesh=mesh,
        compiler_params=pltpu.CompilerParams(use_tc_tiling_on_sc=False),
        out_type=[
            jax.ShapeDtypeStruct((2, N_PAD, D_HALF), jnp.float32),
            jax.ShapeDtypeStruct((2, N_PAD, D_HALF), jnp.float32),
            jax.ShapeDtypeStruct((2, N_PAD, D_HALF), jnp.float32),
            jax.ShapeDtypeStruct((2, N_PAD, D_HALF), jnp.float32),
        ],
        scratch_types=(
            [pltpu.VMEM_SHARED((N_PAD, D_HALF), jnp.float32)] * 2
            + [pltpu.VMEM((2, CHUNK_E), jnp.int32)] * 2
            + [pltpu.VMEM((CHUNK_E,), jnp.float32)] * 2
            + [pltpu.VMEM((CHUNK_E, D_HALF), jnp.float32)] * 4
            + [pltpu.VMEM((CHUNK_E,), jnp.int32)] * 2
            + [pltpu.VMEM((CHUNK_E,), jnp.int32)] * 4
            + [pltpu.VMEM((ROW_CHUNK, D_HALF), jnp.float32)] * 7
            + [pltpu.VMEM((16,), jnp.float32)]
            + [pltpu.SemaphoreType.DMA] * 8
        ),
    )
    return f(x_pair, epack, wpack, da)


def kernel(x, edge_index, edge_weight, delta, a):
    x_pair = x.reshape(N_NODES, 2, D_HALF).transpose(1, 0, 2)
    x_pair = jnp.pad(x_pair, ((0, 0), (0, N_PAD - N_NODES), (0, 0)))
    pad = E_PAD - N_EDGES
    src = jnp.pad(edge_index[1], (0, pad))
    dst = jnp.pad(edge_index[0], (0, pad))
    epack = jnp.stack([src, dst], axis=0)
    epack = epack.reshape(2, N_SUBCORES * N_CHUNKS, CHUNK_E).transpose(1, 0, 2)
    wpack = jnp.pad(edge_weight, (0, pad)).reshape(N_SUBCORES * N_CHUNKS, CHUNK_E)
    da = jnp.concatenate([delta, a, jnp.zeros((14,), jnp.float32)])
    low_p, high_p, _tx1, _tx2 = _run(x_pair, epack, wpack, da)
    low = low_p[:, :N_NODES].transpose(1, 0, 2).reshape(N_NODES, D_FEAT)
    high = high_p[:, :N_NODES].transpose(1, 0, 2).reshape(N_NODES, D_FEAT)
    return (low, high)


# speedup vs baseline: 1.0641x; 1.0641x over previous
import jax
import jax.numpy as jnp
from jax import lax
from jax.experimental import pallas as pl
from jax.experimental.pallas import tpu as pltpu
from jax.experimental.pallas import tpu_sc as plsc

N_NODES = 10000
N_EDGES = 320000
D_FEAT = 128
D_HALF = 64

N_PAD = 10240
N_SUBCORES = 16
ROWS_PER_TILE = N_PAD // N_SUBCORES
CHUNK_E = 128
N_CHUNKS = 160
E_PAD = N_CHUNKS * CHUNK_E * N_SUBCORES
ROW_CHUNK = 32
N_ROW_CHUNKS = ROWS_PER_TILE // ROW_CHUNK


def _zero_rows(zbuf, buf, r0):
    def body(q, _):
        pltpu.sync_copy(zbuf, buf.at[pl.ds(r0 + q * ROW_CHUNK, ROW_CHUNK)])
        return _

    lax.fori_loop(0, N_ROW_CHUNKS, body, None)


def _zero_fill(ref, n_rows):
    zv = jnp.zeros((16,), jnp.float32)

    def body(j, _):
        for d in range(D_HALF // 16):
            ref[j, pl.ds(16 * d, 16)] = zv
        return _

    lax.fori_loop(0, n_rows, body, None)


def _scale_rows(wbuf, rows, scaled):
    def group_body(g, _):
        wgrp = wbuf[pl.ds(g * 16, 16)]
        for j16 in range(16):
            j = g * 16 + j16
            wvec = wgrp.at[jnp.full((16,), j16, jnp.int32)].get(
                mode="promise_in_bounds")
            for d in range(D_HALF // 16):
                sl = pl.ds(16 * d, 16)
                scaled[j, sl] = rows[j, sl] * wvec
        return _

    lax.fori_loop(0, CHUNK_E // 16, group_body, None)


def _copy_idx(ebuf, srcc, dstc):
    def body(q, _):
        sl = pl.ds(16 * q, 16)
        srcc[sl] = ebuf[0, sl]
        dstc[sl] = ebuf[1, sl]
        return _

    lax.fori_loop(0, CHUNK_E // 16, body, None)


def _hop(src_buf, acc_buf, epack, wpack, sid, bufs):
    (ebuf, wbuf, rows, scaled, srcc, dstc, se, sw, sg, ss) = bufs
    c_base = sid * N_CHUNKS

    def load_e(s, c):
        return pltpu.async_copy(epack.at[c_base + c], ebuf[s], se[s])

    def load_w(s, c):
        return pltpu.async_copy(wpack.at[c_base + c], wbuf[s], sw[s])

    def stage_a(sr, sd, c):
        pltpu.make_async_copy(epack.at[c_base], ebuf[sr], se[sr]).wait()
        _copy_idx(ebuf[sr], srcc[sr], dstc[sd])
        load_e(sr, jnp.minimum(c + 2, N_CHUNKS - 2 + sr))
        pltpu.async_copy(src_buf.at[srcc[sr]], rows[sr], sg[sr])

    def stage_b(sr, sd, c, first):
        pltpu.make_async_copy(src_buf.at[srcc[sr]], rows[sr], sg[sr]).wait()
        pltpu.make_async_copy(wpack.at[c_base], wbuf[sr], sw[sr]).wait()
        if not first:
            pltpu.make_async_copy(scaled[sr], acc_buf.at[dstc[sd]],
                                  ss[sr]).wait()
        _scale_rows(wbuf[sr], rows[sr], scaled[sr])
        load_w(sr, jnp.minimum(c + 2, N_CHUNKS - 2 + sr))
        pltpu.async_copy(scaled[sr], acc_buf.at[dstc[sd]], ss[sr], add=True)

    load_e(0, 0)
    load_w(0, 0)
    load_e(1, 1)
    load_w(1, 1)
    stage_a(0, 0, 0)
    stage_a(1, 1, 1)
    stage_b(0, 0, 0, first=True)
    stage_a(0, 2, 2)
    stage_b(1, 1, 1, first=True)
    stage_a(1, 3, 3)
    stage_b(0, 2, 2, first=False)

    def body(t, _):
        c0 = t * 4
        stage_a(0, 0, c0)
        stage_b(1, 3, c0 - 1, first=False)
        stage_a(1, 1, c0 + 1)
        stage_b(0, 0, c0, first=False)
        stage_a(0, 2, c0 + 2)
        stage_b(1, 1, c0 + 1, first=False)
        stage_a(1, 3, c0 + 3)
        stage_b(0, 2, c0 + 2, first=False)
        return _

    lax.fori_loop(1, N_CHUNKS // 4, body, None)

    stage_b(1, 3, N_CHUNKS - 1, first=False)
    for s in range(2):
        pltpu.make_async_copy(scaled[s], acc_buf.at[dstc[s]], ss[s]).wait()
        pltpu.make_async_copy(epack.at[c_base], ebuf[s], se[s]).wait()
        pltpu.make_async_copy(wpack.at[c_base], wbuf[s], sw[s]).wait()


def _combine_chunk(t0c, t1c, t2c, t3c, lowc, highc, coefs):
    c2l, c1l, c0l, c2h, c1h, c0h = coefs

    def body(j, _):
        for d in range(D_HALF // 16):
            sl = pl.ds(16 * d, 16)
            v0 = t0c[j, sl]
            v1 = t1c[j, sl]
            v2 = t2c[j, sl]
            v3 = t3c[j, sl]
            lowc[j, sl] = v3 + c2l * v2 + c1l * v1 + c0l * v0
            highc[j, sl] = v3 + c2h * v2 + c1h * v1 + c0h * v0
        return _

    lax.fori_loop(0, ROW_CHUNK, body, None)


def _filter_kernel(x_pair, epack, wpack, da_hbm,
                   low_hbm, high_hbm, tx1_hbm, tx2_hbm,
                   buf_a, buf_b,
                   e0, e1, w0, w1, r0b, r1b, p0b, p1b,
                   sc0, sc1, dc0, dc1, dc2, dc3,
                   zbuf, t0c, t1c, t2c, t3c, lowc, highc, dav,
                   se0, se1, sw0, sw1, sg0, sg1, ss0, ss1):
    cid = lax.axis_index("c")
    sid = lax.axis_index("s")
    r0 = sid * ROWS_PER_TILE
    bufs = ((e0, e1), (w0, w1), (r0b, r1b), (p0b, p1b),
            (sc0, sc1), (dc0, dc1, dc2, dc3),
            (se0, se1), (sw0, sw1), (sg0, sg1), (ss0, ss1))

    pltpu.sync_copy(x_pair.at[cid, pl.ds(r0, ROWS_PER_TILE)],
                    buf_a.at[pl.ds(r0, ROWS_PER_TILE)])
    pltpu.sync_copy(da_hbm, dav)
    _zero_fill(zbuf, ROW_CHUNK)
    _zero_rows(zbuf, buf_b, r0)
    plsc.subcore_barrier()

    _hop(buf_a, buf_b, epack, wpack, sid, bufs)
    plsc.subcore_barrier()

    pltpu.sync_copy(buf_b.at[pl.ds(r0, ROWS_PER_TILE)],
                    tx1_hbm.at[cid, pl.ds(r0, ROWS_PER_TILE)])
    _zero_rows(zbuf, buf_a, r0)
    plsc.subcore_barrier()

    _hop(buf_b, buf_a, epack, wpack, sid, bufs)
    plsc.subcore_barrier()

    pltpu.sync_copy(buf_a.at[pl.ds(r0, ROWS_PER_TILE)],
                    tx2_hbm.at[cid, pl.ds(r0, ROWS_PER_TILE)])
    _zero_rows(zbuf, buf_b, r0)
    plsc.subcore_barrier()

    _hop(buf_a, buf_b, epack, wpack, sid, bufs)
    plsc.subcore_barrier()

    davec = dav[pl.ds(0, 16)]
    d = davec[0]
    av = davec[1]
    d2 = d * d
    c2l = -3.0 * d - av
    c1l = 3.0 * d2 + 2.0 * d * av
    c0l = -(d2 * d + d2 * av)
    c2h = -3.0 * d + av
    c1h = 3.0 * d2 - 2.0 * d * av
    c0h = d2 * av - d2 * d
    coefs = (c2l, c1l, c0l, c2h, c1h, c0h)

    def final_body(q, _):
        rq = r0 + q * ROW_CHUNK
        pltpu.sync_copy(x_pair.at[cid, pl.ds(rq, ROW_CHUNK)], t0c)
        pltpu.sync_copy(tx1_hbm.at[cid, pl.ds(rq, ROW_CHUNK)], t1c)
        pltpu.sync_copy(tx2_hbm.at[cid, pl.ds(rq, ROW_CHUNK)], t2c)
        pltpu.sync_copy(buf_b.at[pl.ds(rq, ROW_CHUNK)], t3c)
        _combine_chunk(t0c, t1c, t2c, t3c, lowc, highc, coefs)
        pltpu.sync_copy(lowc, low_hbm.at[cid, pl.ds(rq, ROW_CHUNK)])
        pltpu.sync_copy(highc, high_hbm.at[cid, pl.ds(rq, ROW_CHUNK)])
        return _

    lax.fori_loop(0, N_ROW_CHUNKS, final_body, None)


@jax.jit
def _run(x_pair, epack, wpack, da):
    mesh = plsc.VectorSubcoreMesh(core_axis_name="c", subcore_axis_name="s")
    f = pl.kernel(
        _filter_kernel,
        mesh=mesh,
        compiler_params=pltpu.CompilerParams(use_tc_tiling_on_sc=False),
        out_type=[
            jax.ShapeDtypeStruct((2, N_PAD, D_HALF), jnp.float32),
            jax.ShapeDtypeStruct((2, N_PAD, D_HALF), jnp.float32),
            jax.ShapeDtypeStruct((2, N_PAD, D_HALF), jnp.float32),
            jax.ShapeDtypeStruct((2, N_PAD, D_HALF), jnp.float32),
        ],
        scratch_types=(
            [pltpu.VMEM_SHARED((N_PAD, D_HALF), jnp.float32)] * 2
            + [pltpu.VMEM((2, CHUNK_E), jnp.int32)] * 2
            + [pltpu.VMEM((CHUNK_E,), jnp.float32)] * 2
            + [pltpu.VMEM((CHUNK_E, D_HALF), jnp.float32)] * 4
            + [pltpu.VMEM((CHUNK_E,), jnp.int32)] * 2
            + [pltpu.VMEM((CHUNK_E,), jnp.int32)] * 4
            + [pltpu.VMEM((ROW_CHUNK, D_HALF), jnp.float32)] * 7
            + [pltpu.VMEM((16,), jnp.float32)]
            + [pltpu.SemaphoreType.DMA] * 8
        ),
    )
    return f(x_pair, epack, wpack, da)


def kernel(x, edge_index, edge_weight, delta, a):
    x_pair = x.reshape(N_NODES, 2, D_HALF).transpose(1, 0, 2)
    x_pair = jnp.pad(x_pair, ((0, 0), (0, N_PAD - N_NODES), (0, 0)))
    pad = E_PAD - N_EDGES
    src = jnp.pad(edge_index[1], (0, pad))
    dst = jnp.pad(edge_index[0], (0, pad))
    epack = jnp.stack([src, dst], axis=0)
    epack = epack.reshape(2, N_SUBCORES * N_CHUNKS, CHUNK_E).transpose(1, 0, 2)
    wpack = jnp.pad(edge_weight, (0, pad)).reshape(N_SUBCORES * N_CHUNKS, CHUNK_E)
    da = jnp.concatenate([delta, a, jnp.zeros((14,), jnp.float32)])
    low_p, high_p, _tx1, _tx2 = _run(x_pair, epack, wpack, da)
    low = low_p[:, :N_NODES].transpose(1, 0, 2).reshape(N_NODES, D_FEAT)
    high = high_p[:, :N_NODES].transpose(1, 0, 2).reshape(N_NODES, D_FEAT)
    return (low, high)
